# baseline (speedup 1.0000x reference)
import functools
import math

import jax
import jax.numpy as jnp
import numpy as np
from jax.experimental import pallas as pl
from jax.experimental.pallas import tpu as pltpu

_BN_S = float(1.0 / np.sqrt(1.0 + 1e-5))


def _fold(layers):
    out = []
    for (W, g, b) in layers:
        out.append((W.T * (g * _BN_S)[None, :], b))
    return out


def _mlp_kernel(x_ref, *refs, nlayers, pool):
    h = x_ref[...]
    for i in range(nlayers):
        w = refs[2 * i][...]
        b = refs[2 * i + 1][...]
        h = jnp.dot(h, w, preferred_element_type=jnp.float32)
        h = jnp.maximum(h + b, 0.0)
    o_ref = refs[2 * nlayers]
    if pool > 1:
        r, c = h.shape
        h = jnp.max(h.reshape(r // pool, pool, c), axis=1)
    o_ref[...] = h


def _mlp_pallas(x, folded, tile=512, pool=1):
    N, Cin = x.shape
    nlayers = len(folded)
    Cout = folded[-1][0].shape[1]
    assert N % tile == 0 and tile % pool == 0
    in_specs = [pl.BlockSpec((tile, Cin), lambda i: (i, 0))]
    args = [x]
    for (Wt, b) in folded:
        ci, co = Wt.shape
        in_specs.append(pl.BlockSpec((ci, co), lambda i: (0, 0)))
        in_specs.append(pl.BlockSpec((1, co), lambda i: (0, 0)))
        args.append(Wt)
        args.append(b[None, :])
    out = pl.pallas_call(
        functools.partial(_mlp_kernel, nlayers=nlayers, pool=pool),
        grid=(N // tile,),
        in_specs=in_specs,
        out_specs=pl.BlockSpec((tile // pool, Cout), lambda i: (i, 0)),
        out_shape=jax.ShapeDtypeStruct((N // pool, Cout), jnp.float32),
    )(*args)
    return out


def _head_kernel(x_ref, w_ref, b_ref, o_ref):
    z = jnp.dot(x_ref[...], w_ref[...], preferred_element_type=jnp.float32)
    z = z + b_ref[...]
    m = jnp.max(z, axis=-1, keepdims=True)
    lse = jnp.log(jnp.sum(jnp.exp(z - m), axis=-1, keepdims=True))
    o_ref[...] = z - m - lse


def _head_pallas(x, Wb, tile=1024):
    W, b = Wb
    N, Cin = x.shape
    Cout = W.shape[0]
    out = pl.pallas_call(
        _head_kernel,
        grid=(N // tile,),
        in_specs=[
            pl.BlockSpec((tile, Cin), lambda i: (i, 0)),
            pl.BlockSpec((Cin, Cout), lambda i: (0, 0)),
            pl.BlockSpec((1, Cout), lambda i: (0, 0)),
        ],
        out_specs=pl.BlockSpec((tile, Cout), lambda i: (i, 0)),
        out_shape=jax.ShapeDtypeStruct((N, Cout), jnp.float32),
    )(x, W.T, b[None, :])
    return out


def _sqdist(a, b):
    return (jnp.sum(a * a, -1)[:, :, None] + jnp.sum(b * b, -1)[:, None, :]
            - 2.0 * jnp.einsum('bsd,bnd->bsn', a, b))


def _fps(xyz, npoint):
    B, N, _ = xyz.shape
    def body(i, state):
        dists, inds, far = state
        inds = inds.at[:, i].set(far)
        centroid = jnp.take_along_axis(xyz, far[:, None, None], axis=1)
        d = jnp.sum((xyz - centroid) ** 2, -1)
        dists = jnp.minimum(dists, d)
        far = jnp.argmax(dists, -1).astype(jnp.int32)
        return (dists, inds, far)
    state = (jnp.full((B, N), 1e10, jnp.float32),
             jnp.zeros((B, npoint), jnp.int32),
             jnp.zeros((B,), jnp.int32))
    _, inds, _ = jax.lax.fori_loop(0, npoint, body, state)
    return inds


def _ball_query(radius, nsample, xyz, new_xyz):
    B, S, _ = new_xyz.shape
    N = xyz.shape[1]
    d = _sqdist(new_xyz, xyz)
    gi = jnp.broadcast_to(jnp.arange(N, dtype=jnp.int32)[None, None, :], (B, S, N))
    gi = jnp.where(d > radius * radius, N, gi)
    gi = jnp.sort(gi, axis=-1)[:, :, :nsample]
    first = gi[:, :, :1]
    return jnp.where(gi == N, first, gi)


def _index_points(points, idx):
    return jax.vmap(lambda p, i: p[i])(points, idx)


def _sa1(xyz, feats, radius, nsample, layers):
    inds = _fps(xyz, 8192)
    new_xyz = _index_points(xyz, inds)
    idx = _ball_query(radius, nsample, xyz, new_xyz)
    g_feats = _index_points(feats, idx)
    g_xyz = (_index_points(xyz, idx) - new_xyz[:, :, None, :]) / radius
    g = jnp.concatenate([g_xyz, g_feats], -1)
    S = g.shape[1]
    folded = _fold(layers)
    h = _mlp_pallas(g.reshape(S * nsample, -1), folded, tile=32 * nsample,
                    pool=nsample)
    return new_xyz, h[None], inds


def _sa_noxyz(xyz, feats, npoint, radius, nsample, layers):
    inds = _fps(xyz, npoint)
    new_xyz = _index_points(xyz, inds)
    idx = _ball_query(radius, nsample, xyz, new_xyz)
    folded = _fold(layers)
    H = _mlp_pallas(feats[0], folded, tile=512)
    g = _index_points(H[None], idx)
    pooled = jnp.max(g, axis=2)
    return new_xyz, pooled, inds


def _fp(unk_xyz, kn_xyz, skip, kn_feats, layers):
    d = _sqdist(unk_xyz, kn_xyz)
    negd, idx = jax.lax.top_k(-d, 3)
    dist = jnp.maximum(-negd, 0.0)
    w = 1.0 / (dist + 1e-8)
    w = w / jnp.sum(w, -1, keepdims=True)
    interp = jnp.sum(_index_points(kn_feats, idx) * w[..., None], axis=2)
    h = jnp.concatenate([interp, skip], -1)
    folded = _fold(layers)
    return _mlp_pallas(h[0], folded, tile=512)[None]


def kernel(pointcloud, params):
    xyz = pointcloud[..., 0:3]
    feats = pointcloud[..., 3:]
    local = _mlp_pallas(feats[0], _fold(params['local_feature']), tile=512)[None]
    local_prob = _head_pallas(local[0], params['local_predictor'])[None]

    x1, f1, i1 = _sa1(xyz, feats, 1.0, 16, params['sa1'])
    x2, f2, i2 = _sa_noxyz(x1, f1, 4096, 5.0, 64, params['sa2'])
    x3, f3, i3 = _sa_noxyz(x2, f2, 2048, 15.0, 64, params['sa3'])

    f = _fp(x2, x3, f2, f3, params['fp2'])
    f = _fp(x1, x2, f1, f, params['fp3'])
    skip = jnp.concatenate([xyz, feats], -1)
    f = _fp(xyz, x1, skip, f, params['fp4'])

    g = _mlp_pallas(f[0], _fold(params['global_feature']), tile=512)[None]
    global_prob = _head_pallas(g[0], params['global_predictor'])[None]

    h = jnp.concatenate([local, g], -1)
    h = _mlp_pallas(h[0], _fold([params['lgp_conv']]), tile=512)[None]
    pred = _head_pallas(h[0], params['lgp_out'])[None]
    return (pred, local_prob, global_prob)


# speedup vs baseline: 1.9944x; 1.9944x over previous
import functools
import math

import jax
import jax.numpy as jnp
import numpy as np
from jax.experimental import pallas as pl
from jax.experimental.pallas import tpu as pltpu

_BN_S = float(1.0 / np.sqrt(1.0 + 1e-5))


def _fold(layers):
    out = []
    for (W, g, b) in layers:
        out.append((W.T * (g * _BN_S)[None, :], b))
    return out


def _mlp_kernel(x_ref, *refs, nlayers, pool):
    h = x_ref[...]
    for i in range(nlayers):
        w = refs[2 * i][...]
        b = refs[2 * i + 1][...]
        h = jnp.dot(h, w, preferred_element_type=jnp.float32)
        h = jnp.maximum(h + b, 0.0)
    o_ref = refs[2 * nlayers]
    if pool > 1:
        r, c = h.shape
        h = jnp.max(h.reshape(r // pool, pool, c), axis=1)
    o_ref[...] = h


def _mlp_pallas(x, folded, tile=512, pool=1):
    N, Cin = x.shape
    nlayers = len(folded)
    Cout = folded[-1][0].shape[1]
    assert N % tile == 0 and tile % pool == 0
    in_specs = [pl.BlockSpec((tile, Cin), lambda i: (i, 0))]
    args = [x]
    for (Wt, b) in folded:
        ci, co = Wt.shape
        in_specs.append(pl.BlockSpec((ci, co), lambda i: (0, 0)))
        in_specs.append(pl.BlockSpec((1, co), lambda i: (0, 0)))
        args.append(Wt)
        args.append(b[None, :])
    out = pl.pallas_call(
        functools.partial(_mlp_kernel, nlayers=nlayers, pool=pool),
        grid=(N // tile,),
        in_specs=in_specs,
        out_specs=pl.BlockSpec((tile // pool, Cout), lambda i: (i, 0)),
        out_shape=jax.ShapeDtypeStruct((N // pool, Cout), jnp.float32),
    )(*args)
    return out


def _head_kernel(x_ref, w_ref, b_ref, o_ref):
    z = jnp.dot(x_ref[...], w_ref[...], preferred_element_type=jnp.float32)
    z = z + b_ref[...]
    m = jnp.max(z, axis=-1, keepdims=True)
    lse = jnp.log(jnp.sum(jnp.exp(z - m), axis=-1, keepdims=True))
    o_ref[...] = z - m - lse


def _head_pallas(x, Wb, tile=1024):
    W, b = Wb
    N, Cin = x.shape
    Cout = W.shape[0]
    out = pl.pallas_call(
        _head_kernel,
        grid=(N // tile,),
        in_specs=[
            pl.BlockSpec((tile, Cin), lambda i: (i, 0)),
            pl.BlockSpec((Cin, Cout), lambda i: (0, 0)),
            pl.BlockSpec((1, Cout), lambda i: (0, 0)),
        ],
        out_specs=pl.BlockSpec((tile, Cout), lambda i: (i, 0)),
        out_shape=jax.ShapeDtypeStruct((N, Cout), jnp.float32),
    )(x, W.T, b[None, :])
    return out


def _sqdist(a, b):
    return (jnp.sum(a * a, -1)[:, :, None] + jnp.sum(b * b, -1)[:, None, :]
            - 2.0 * jnp.einsum('bsd,bnd->bsn', a, b))


def _fps_kernel(x_ref, y_ref, z_ref, o_ref, d_ref, *, n, npoint):
    R, C = x_ref.shape
    iota = (jax.lax.broadcasted_iota(jnp.int32, (R, C), 0) * C
            + jax.lax.broadcasted_iota(jnp.int32, (R, C), 1))
    oR, oC = o_ref.shape
    oiota = (jax.lax.broadcasted_iota(jnp.int32, (oR, oC), 0) * oC
             + jax.lax.broadcasted_iota(jnp.int32, (oR, oC), 1))
    d_ref[...] = jnp.full((R, C), 1e10, jnp.float32)
    o_ref[...] = jnp.zeros((oR, oC), jnp.int32)

    def body(i, far):
        o_ref[...] += jnp.where(oiota == i, far, 0)
        x = x_ref[...]
        y = y_ref[...]
        z = z_ref[...]
        sel = iota == far
        cx = jnp.sum(jnp.where(sel, x, 0.0))
        cy = jnp.sum(jnp.where(sel, y, 0.0))
        cz = jnp.sum(jnp.where(sel, z, 0.0))
        dx = x - cx
        dy = y - cy
        dz = z - cz
        d = (dx * dx + dy * dy) + dz * dz
        nd = jnp.minimum(d_ref[...], d)
        d_ref[...] = nd
        m = jnp.max(nd)
        return jnp.min(jnp.where(nd == m, iota, n)).astype(jnp.int32)

    jax.lax.fori_loop(0, npoint, body, jnp.int32(0))


def _fps(xyz, npoint):
    B, N, _ = xyz.shape
    x = xyz[0, :, 0].reshape(8, N // 8)
    y = xyz[0, :, 1].reshape(8, N // 8)
    z = xyz[0, :, 2].reshape(8, N // 8)
    out = pl.pallas_call(
        functools.partial(_fps_kernel, n=N, npoint=npoint),
        in_specs=[pl.BlockSpec((8, N // 8), lambda: (0, 0))] * 3,
        out_specs=pl.BlockSpec((8, npoint // 8), lambda: (0, 0)),
        out_shape=jax.ShapeDtypeStruct((8, npoint // 8), jnp.int32),
        scratch_shapes=[pltpu.VMEM((8, N // 8), jnp.float32)],
    )(x, y, z)
    return out.reshape(1, npoint)


def _ball_query(radius, nsample, xyz, new_xyz):
    B, S, _ = new_xyz.shape
    N = xyz.shape[1]
    d = _sqdist(new_xyz, xyz)
    gi = jnp.broadcast_to(jnp.arange(N, dtype=jnp.int32)[None, None, :], (B, S, N))
    gi = jnp.where(d > radius * radius, N, gi)
    gi = jnp.sort(gi, axis=-1)[:, :, :nsample]
    first = gi[:, :, :1]
    return jnp.where(gi == N, first, gi)


def _index_points(points, idx):
    return jax.vmap(lambda p, i: p[i])(points, idx)


def _sa1(xyz, feats, radius, nsample, layers):
    inds = _fps(xyz, 8192)
    new_xyz = _index_points(xyz, inds)
    idx = _ball_query(radius, nsample, xyz, new_xyz)
    g_feats = _index_points(feats, idx)
    g_xyz = (_index_points(xyz, idx) - new_xyz[:, :, None, :]) / radius
    g = jnp.concatenate([g_xyz, g_feats], -1)
    S = g.shape[1]
    folded = _fold(layers)
    h = _mlp_pallas(g.reshape(S * nsample, -1), folded, tile=32 * nsample,
                    pool=nsample)
    return new_xyz, h[None], inds


def _sa_noxyz(xyz, feats, npoint, radius, nsample, layers):
    inds = _fps(xyz, npoint)
    new_xyz = _index_points(xyz, inds)
    idx = _ball_query(radius, nsample, xyz, new_xyz)
    folded = _fold(layers)
    H = _mlp_pallas(feats[0], folded, tile=512)
    g = _index_points(H[None], idx)
    pooled = jnp.max(g, axis=2)
    return new_xyz, pooled, inds


def _fp(unk_xyz, kn_xyz, skip, kn_feats, layers):
    d = _sqdist(unk_xyz, kn_xyz)
    negd, idx = jax.lax.top_k(-d, 3)
    dist = jnp.maximum(-negd, 0.0)
    w = 1.0 / (dist + 1e-8)
    w = w / jnp.sum(w, -1, keepdims=True)
    interp = jnp.sum(_index_points(kn_feats, idx) * w[..., None], axis=2)
    h = jnp.concatenate([interp, skip], -1)
    folded = _fold(layers)
    return _mlp_pallas(h[0], folded, tile=512)[None]


def kernel(pointcloud, params):
    xyz = pointcloud[..., 0:3]
    feats = pointcloud[..., 3:]
    local = _mlp_pallas(feats[0], _fold(params['local_feature']), tile=512)[None]
    local_prob = _head_pallas(local[0], params['local_predictor'])[None]

    x1, f1, i1 = _sa1(xyz, feats, 1.0, 16, params['sa1'])
    x2, f2, i2 = _sa_noxyz(x1, f1, 4096, 5.0, 64, params['sa2'])
    x3, f3, i3 = _sa_noxyz(x2, f2, 2048, 15.0, 64, params['sa3'])

    f = _fp(x2, x3, f2, f3, params['fp2'])
    f = _fp(x1, x2, f1, f, params['fp3'])
    skip = jnp.concatenate([xyz, feats], -1)
    f = _fp(xyz, x1, skip, f, params['fp4'])

    g = _mlp_pallas(f[0], _fold(params['global_feature']), tile=512)[None]
    global_prob = _head_pallas(g[0], params['global_predictor'])[None]

    h = jnp.concatenate([local, g], -1)
    h = _mlp_pallas(h[0], _fold([params['lgp_conv']]), tile=512)[None]
    pred = _head_pallas(h[0], params['lgp_out'])[None]
    return (pred, local_prob, global_prob)


# speedup vs baseline: 1.9975x; 1.0016x over previous
import functools
import math

import jax
import jax.numpy as jnp
import numpy as np
from jax.experimental import pallas as pl
from jax.experimental.pallas import tpu as pltpu

_BN_S = float(1.0 / np.sqrt(1.0 + 1e-5))


def _fold(layers):
    out = []
    for (W, g, b) in layers:
        out.append((W.T * (g * _BN_S)[None, :], b))
    return out


def _mlp_kernel(x_ref, *refs, nlayers, pool):
    h = x_ref[...]
    for i in range(nlayers):
        w = refs[2 * i][...]
        b = refs[2 * i + 1][...]
        h = jnp.dot(h, w, preferred_element_type=jnp.float32)
        h = jnp.maximum(h + b, 0.0)
    o_ref = refs[2 * nlayers]
    if pool > 1:
        r, c = h.shape
        h = jnp.max(h.reshape(r // pool, pool, c), axis=1)
    o_ref[...] = h


def _mlp_pallas(x, folded, tile=512, pool=1):
    N, Cin = x.shape
    nlayers = len(folded)
    Cout = folded[-1][0].shape[1]
    assert N % tile == 0 and tile % pool == 0
    in_specs = [pl.BlockSpec((tile, Cin), lambda i: (i, 0))]
    args = [x]
    for (Wt, b) in folded:
        ci, co = Wt.shape
        in_specs.append(pl.BlockSpec((ci, co), lambda i: (0, 0)))
        in_specs.append(pl.BlockSpec((1, co), lambda i: (0, 0)))
        args.append(Wt)
        args.append(b[None, :])
    out = pl.pallas_call(
        functools.partial(_mlp_kernel, nlayers=nlayers, pool=pool),
        grid=(N // tile,),
        in_specs=in_specs,
        out_specs=pl.BlockSpec((tile // pool, Cout), lambda i: (i, 0)),
        out_shape=jax.ShapeDtypeStruct((N // pool, Cout), jnp.float32),
    )(*args)
    return out


def _head_kernel(x_ref, w_ref, b_ref, o_ref):
    z = jnp.dot(x_ref[...], w_ref[...], preferred_element_type=jnp.float32)
    z = z + b_ref[...]
    m = jnp.max(z, axis=-1, keepdims=True)
    lse = jnp.log(jnp.sum(jnp.exp(z - m), axis=-1, keepdims=True))
    o_ref[...] = z - m - lse


def _head_pallas(x, Wb, tile=1024):
    W, b = Wb
    N, Cin = x.shape
    Cout = W.shape[0]
    out = pl.pallas_call(
        _head_kernel,
        grid=(N // tile,),
        in_specs=[
            pl.BlockSpec((tile, Cin), lambda i: (i, 0)),
            pl.BlockSpec((Cin, Cout), lambda i: (0, 0)),
            pl.BlockSpec((1, Cout), lambda i: (0, 0)),
        ],
        out_specs=pl.BlockSpec((tile, Cout), lambda i: (i, 0)),
        out_shape=jax.ShapeDtypeStruct((N, Cout), jnp.float32),
    )(x, W.T, b[None, :])
    return out


def _sqdist(a, b):
    return (jnp.sum(a * a, -1)[:, :, None] + jnp.sum(b * b, -1)[:, None, :]
            - 2.0 * jnp.einsum('bsd,bnd->bsn', a, b))


def _fps_kernel(x_ref, y_ref, z_ref, o_ref, d_ref, *, n, npoint):
    R, C = x_ref.shape
    iota = (jax.lax.broadcasted_iota(jnp.int32, (R, C), 0) * C
            + jax.lax.broadcasted_iota(jnp.int32, (R, C), 1))
    oR, oC = o_ref.shape
    oiota = (jax.lax.broadcasted_iota(jnp.int32, (oR, oC), 0) * oC
             + jax.lax.broadcasted_iota(jnp.int32, (oR, oC), 1))
    d_ref[...] = jnp.full((R, C), 1e10, jnp.float32)
    o_ref[...] = jnp.zeros((oR, oC), jnp.int32)

    def body(i, far):
        o_ref[...] += jnp.where(oiota == i, far, 0)
        x = x_ref[...]
        y = y_ref[...]
        z = z_ref[...]
        sel = iota == far
        cx = jnp.sum(jnp.where(sel, x, 0.0))
        cy = jnp.sum(jnp.where(sel, y, 0.0))
        cz = jnp.sum(jnp.where(sel, z, 0.0))
        dx = x - cx
        dy = y - cy
        dz = z - cz
        d = (dx * dx + dy * dy) + dz * dz
        nd = jnp.minimum(d_ref[...], d)
        d_ref[...] = nd
        m = jnp.max(nd)
        return jnp.min(jnp.where(nd == m, iota, n)).astype(jnp.int32)

    jax.lax.fori_loop(0, npoint, body, jnp.int32(0))


def _fps(xyz, npoint):
    B, N, _ = xyz.shape
    x = xyz[0, :, 0].reshape(8, N // 8)
    y = xyz[0, :, 1].reshape(8, N // 8)
    z = xyz[0, :, 2].reshape(8, N // 8)
    out = pl.pallas_call(
        functools.partial(_fps_kernel, n=N, npoint=npoint),
        in_specs=[pl.BlockSpec((8, N // 8), lambda: (0, 0))] * 3,
        out_specs=pl.BlockSpec((8, npoint // 8), lambda: (0, 0)),
        out_shape=jax.ShapeDtypeStruct((8, npoint // 8), jnp.int32),
        scratch_shapes=[pltpu.VMEM((8, N // 8), jnp.float32)],
    )(x, y, z)
    return out.reshape(1, npoint)


def _ball_query(radius, nsample, xyz, new_xyz):
    B, S, _ = new_xyz.shape
    N = xyz.shape[1]
    d = _sqdist(new_xyz, xyz)
    gi = jnp.broadcast_to(jnp.arange(N, dtype=jnp.int32)[None, None, :], (B, S, N))
    gi = jnp.where(d > radius * radius, N, gi)
    neg = jax.lax.top_k(-gi, nsample)[0]
    gi = -neg
    first = gi[:, :, :1]
    return jnp.where(gi == N, first, gi)


def _index_points(points, idx):
    return jax.vmap(lambda p, i: p[i])(points, idx)


def _sa1(xyz, feats, radius, nsample, layers):
    inds = _fps(xyz, 8192)
    new_xyz = _index_points(xyz, inds)
    idx = _ball_query(radius, nsample, xyz, new_xyz)
    g_feats = _index_points(feats, idx)
    g_xyz = (_index_points(xyz, idx) - new_xyz[:, :, None, :]) / radius
    g = jnp.concatenate([g_xyz, g_feats], -1)
    S = g.shape[1]
    folded = _fold(layers)
    h = _mlp_pallas(g.reshape(S * nsample, -1), folded, tile=32 * nsample,
                    pool=nsample)
    return new_xyz, h[None], inds


def _sa_noxyz(xyz, feats, npoint, radius, nsample, layers):
    inds = _fps(xyz, npoint)
    new_xyz = _index_points(xyz, inds)
    idx = _ball_query(radius, nsample, xyz, new_xyz)
    folded = _fold(layers)
    H = _mlp_pallas(feats[0], folded, tile=512)
    g = _index_points(H[None], idx)
    pooled = jnp.max(g, axis=2)
    return new_xyz, pooled, inds


def _fp(unk_xyz, kn_xyz, skip, kn_feats, layers):
    d = _sqdist(unk_xyz, kn_xyz)
    negd, idx = jax.lax.top_k(-d, 3)
    dist = jnp.maximum(-negd, 0.0)
    w = 1.0 / (dist + 1e-8)
    w = w / jnp.sum(w, -1, keepdims=True)
    interp = jnp.sum(_index_points(kn_feats, idx) * w[..., None], axis=2)
    h = jnp.concatenate([interp, skip], -1)
    folded = _fold(layers)
    return _mlp_pallas(h[0], folded, tile=512)[None]


def kernel(pointcloud, params):
    xyz = pointcloud[..., 0:3]
    feats = pointcloud[..., 3:]
    local = _mlp_pallas(feats[0], _fold(params['local_feature']), tile=512)[None]
    local_prob = _head_pallas(local[0], params['local_predictor'])[None]

    x1, f1, i1 = _sa1(xyz, feats, 1.0, 16, params['sa1'])
    x2, f2, i2 = _sa_noxyz(x1, f1, 4096, 5.0, 64, params['sa2'])
    x3, f3, i3 = _sa_noxyz(x2, f2, 2048, 15.0, 64, params['sa3'])

    f = _fp(x2, x3, f2, f3, params['fp2'])
    f = _fp(x1, x2, f1, f, params['fp3'])
    skip = jnp.concatenate([xyz, feats], -1)
    f = _fp(xyz, x1, skip, f, params['fp4'])

    g = _mlp_pallas(f[0], _fold(params['global_feature']), tile=512)[None]
    global_prob = _head_pallas(g[0], params['global_predictor'])[None]

    h = jnp.concatenate([local, g], -1)
    h = _mlp_pallas(h[0], _fold([params['lgp_conv']]), tile=512)[None]
    pred = _head_pallas(h[0], params['lgp_out'])[None]
    return (pred, local_prob, global_prob)


# speedup vs baseline: 3.1881x; 1.5961x over previous
import functools
import math

import jax
import jax.numpy as jnp
import numpy as np
from jax.experimental import pallas as pl
from jax.experimental.pallas import tpu as pltpu

_BN_S = float(1.0 / np.sqrt(1.0 + 1e-5))


def _fold(layers):
    out = []
    for (W, g, b) in layers:
        out.append((W.T * (g * _BN_S)[None, :], b))
    return out


def _mlp_kernel(x_ref, *refs, nlayers, pool):
    h = x_ref[...]
    for i in range(nlayers):
        w = refs[2 * i][...]
        b = refs[2 * i + 1][...]
        h = jnp.dot(h, w, preferred_element_type=jnp.float32)
        h = jnp.maximum(h + b, 0.0)
    o_ref = refs[2 * nlayers]
    if pool > 1:
        r, c = h.shape
        h = jnp.max(h.reshape(r // pool, pool, c), axis=1)
    o_ref[...] = h


def _mlp_pallas(x, folded, tile=512, pool=1):
    N, Cin = x.shape
    nlayers = len(folded)
    Cout = folded[-1][0].shape[1]
    assert N % tile == 0 and tile % pool == 0
    in_specs = [pl.BlockSpec((tile, Cin), lambda i: (i, 0))]
    args = [x]
    for (Wt, b) in folded:
        ci, co = Wt.shape
        in_specs.append(pl.BlockSpec((ci, co), lambda i: (0, 0)))
        in_specs.append(pl.BlockSpec((1, co), lambda i: (0, 0)))
        args.append(Wt)
        args.append(b[None, :])
    out = pl.pallas_call(
        functools.partial(_mlp_kernel, nlayers=nlayers, pool=pool),
        grid=(N // tile,),
        in_specs=in_specs,
        out_specs=pl.BlockSpec((tile // pool, Cout), lambda i: (i, 0)),
        out_shape=jax.ShapeDtypeStruct((N // pool, Cout), jnp.float32),
    )(*args)
    return out


def _head_kernel(x_ref, w_ref, b_ref, o_ref):
    z = jnp.dot(x_ref[...], w_ref[...], preferred_element_type=jnp.float32)
    z = z + b_ref[...]
    m = jnp.max(z, axis=-1, keepdims=True)
    lse = jnp.log(jnp.sum(jnp.exp(z - m), axis=-1, keepdims=True))
    o_ref[...] = z - m - lse


def _head_pallas(x, Wb, tile=1024):
    W, b = Wb
    N, Cin = x.shape
    Cout = W.shape[0]
    out = pl.pallas_call(
        _head_kernel,
        grid=(N // tile,),
        in_specs=[
            pl.BlockSpec((tile, Cin), lambda i: (i, 0)),
            pl.BlockSpec((Cin, Cout), lambda i: (0, 0)),
            pl.BlockSpec((1, Cout), lambda i: (0, 0)),
        ],
        out_specs=pl.BlockSpec((tile, Cout), lambda i: (i, 0)),
        out_shape=jax.ShapeDtypeStruct((N, Cout), jnp.float32),
    )(x, W.T, b[None, :])
    return out


def _sqdist(a, b):
    return (jnp.sum(a * a, -1)[:, :, None] + jnp.sum(b * b, -1)[:, None, :]
            - 2.0 * jnp.einsum('bsd,bnd->bsn', a, b))


def _fps_kernel(x_ref, y_ref, z_ref, o_ref, d_ref, *, n, npoint):
    R, C = x_ref.shape
    iota = (jax.lax.broadcasted_iota(jnp.int32, (R, C), 0) * C
            + jax.lax.broadcasted_iota(jnp.int32, (R, C), 1))
    oR, oC = o_ref.shape
    oiota = (jax.lax.broadcasted_iota(jnp.int32, (oR, oC), 0) * oC
             + jax.lax.broadcasted_iota(jnp.int32, (oR, oC), 1))
    d_ref[...] = jnp.full((R, C), 1e10, jnp.float32)
    o_ref[...] = jnp.zeros((oR, oC), jnp.int32)

    def body(i, far):
        o_ref[...] += jnp.where(oiota == i, far, 0)
        x = x_ref[...]
        y = y_ref[...]
        z = z_ref[...]
        sel = iota == far
        cx = jnp.sum(jnp.where(sel, x, 0.0))
        cy = jnp.sum(jnp.where(sel, y, 0.0))
        cz = jnp.sum(jnp.where(sel, z, 0.0))
        dx = x - cx
        dy = y - cy
        dz = z - cz
        d = (dx * dx + dy * dy) + dz * dz
        nd = jnp.minimum(d_ref[...], d)
        d_ref[...] = nd
        m = jnp.max(nd)
        return jnp.min(jnp.where(nd == m, iota, n)).astype(jnp.int32)

    jax.lax.fori_loop(0, npoint, body, jnp.int32(0))


def _fps(xyz, npoint):
    B, N, _ = xyz.shape
    x = xyz[0, :, 0].reshape(8, N // 8)
    y = xyz[0, :, 1].reshape(8, N // 8)
    z = xyz[0, :, 2].reshape(8, N // 8)
    out = pl.pallas_call(
        functools.partial(_fps_kernel, n=N, npoint=npoint),
        in_specs=[pl.BlockSpec((8, N // 8), lambda: (0, 0))] * 3,
        out_specs=pl.BlockSpec((8, npoint // 8), lambda: (0, 0)),
        out_shape=jax.ShapeDtypeStruct((8, npoint // 8), jnp.int32),
        scratch_shapes=[pltpu.VMEM((8, N // 8), jnp.float32)],
    )(x, y, z)
    return out.reshape(1, npoint)


def _ball_query(radius, nsample, xyz, new_xyz):
    B, S, _ = new_xyz.shape
    N = xyz.shape[1]
    d = _sqdist(new_xyz, xyz)
    gi = jnp.broadcast_to(jnp.arange(N, dtype=jnp.int32)[None, None, :], (B, S, N))
    gi = jnp.where(d > radius * radius, N, gi)
    neg = jax.lax.top_k(-gi, nsample)[0]
    gi = -neg
    first = gi[:, :, :1]
    return jnp.where(gi == N, first, gi)


def _index_points(points, idx):
    return jax.vmap(lambda p, i: p[i])(points, idx)


def _sa1(xyz, feats, radius, nsample, layers):
    inds = _fps(xyz, 8192)
    new_xyz = _index_points(xyz, inds)
    idx = _ball_query(radius, nsample, xyz, new_xyz)
    g_feats = _index_points(feats, idx)
    g_xyz = (_index_points(xyz, idx) - new_xyz[:, :, None, :]) / radius
    g = jnp.concatenate([g_xyz, g_feats], -1)
    S = g.shape[1]
    folded = _fold(layers)
    h = _mlp_pallas(g.reshape(S * nsample, -1), folded, tile=32 * nsample,
                    pool=nsample)
    return new_xyz, h[None], inds


def _sa_noxyz(xyz, feats, npoint, radius, nsample, layers):
    inds = _fps(xyz, npoint)
    new_xyz = _index_points(xyz, inds)
    idx = _ball_query(radius, nsample, xyz, new_xyz)
    folded = _fold(layers)
    H = _mlp_pallas(feats[0], folded, tile=512)
    g = _index_points(H[None], idx)
    pooled = jnp.max(g, axis=2)
    return new_xyz, pooled, inds


def _knn3_kernel(c_ref, csq_ref, kt_ref, ksq_ref, di_ref, ii_ref, *, n):
    c = c_ref[...]
    cross = jnp.dot(c, kt_ref[...], preferred_element_type=jnp.float32)
    d = (csq_ref[...] + ksq_ref[...]) - 2.0 * cross
    iota = jax.lax.broadcasted_iota(jnp.int32, d.shape, 1)
    for t in range(3):
        mt = jnp.min(d, axis=1, keepdims=True)
        it = jnp.min(jnp.where(d == mt, iota, n), axis=1, keepdims=True)
        di_ref[:, t:t + 1] = mt
        ii_ref[:, t:t + 1] = it
        d = jnp.where(iota == it, jnp.float32(np.inf), d)


def _knn3(unk, kn, tile=128):
    S = unk.shape[0]
    N = kn.shape[0]
    csq = jnp.sum(unk * unk, -1)[:, None]
    ksq = jnp.sum(kn * kn, -1)[None, :]
    d, idx = pl.pallas_call(
        functools.partial(_knn3_kernel, n=N),
        grid=(S // tile,),
        in_specs=[
            pl.BlockSpec((tile, 3), lambda i: (i, 0)),
            pl.BlockSpec((tile, 1), lambda i: (i, 0)),
            pl.BlockSpec((3, N), lambda i: (0, 0)),
            pl.BlockSpec((1, N), lambda i: (0, 0)),
        ],
        out_specs=[pl.BlockSpec((tile, 3), lambda i: (i, 0))] * 2,
        out_shape=[jax.ShapeDtypeStruct((S, 3), jnp.float32),
                   jax.ShapeDtypeStruct((S, 3), jnp.int32)],
    )(unk, csq, kn.T, ksq)
    return d, idx


def _fp(unk_xyz, kn_xyz, skip, kn_feats, layers):
    d3, i3 = _knn3(unk_xyz[0], kn_xyz[0])
    idx = i3[None]
    dist = jnp.maximum(d3, 0.0)[None]
    w = 1.0 / (dist + 1e-8)
    w = w / jnp.sum(w, -1, keepdims=True)
    interp = jnp.sum(_index_points(kn_feats, idx) * w[..., None], axis=2)
    h = jnp.concatenate([interp, skip], -1)
    folded = _fold(layers)
    return _mlp_pallas(h[0], folded, tile=512)[None]


def kernel(pointcloud, params):
    xyz = pointcloud[..., 0:3]
    feats = pointcloud[..., 3:]
    local = _mlp_pallas(feats[0], _fold(params['local_feature']), tile=512)[None]
    local_prob = _head_pallas(local[0], params['local_predictor'])[None]

    x1, f1, i1 = _sa1(xyz, feats, 1.0, 16, params['sa1'])
    x2, f2, i2 = _sa_noxyz(x1, f1, 4096, 5.0, 64, params['sa2'])
    x3, f3, i3 = _sa_noxyz(x2, f2, 2048, 15.0, 64, params['sa3'])

    f = _fp(x2, x3, f2, f3, params['fp2'])
    f = _fp(x1, x2, f1, f, params['fp3'])
    skip = jnp.concatenate([xyz, feats], -1)
    f = _fp(xyz, x1, skip, f, params['fp4'])

    g = _mlp_pallas(f[0], _fold(params['global_feature']), tile=512)[None]
    global_prob = _head_pallas(g[0], params['global_predictor'])[None]

    h = jnp.concatenate([local, g], -1)
    h = _mlp_pallas(h[0], _fold([params['lgp_conv']]), tile=512)[None]
    pred = _head_pallas(h[0], params['lgp_out'])[None]
    return (pred, local_prob, global_prob)
